# baseline (speedup 1.0000x reference)
import functools

import jax
import jax.numpy as jnp
from jax import lax
from jax.experimental import pallas as pl
from jax.experimental.pallas import tpu as pltpu
from jax.experimental.pallas import tpu_sc as plsc

_L = 16
_W = 8
_G = 32


@functools.lru_cache(maxsize=None)
def _make_gather(V, D, B):
    info = plsc.get_sparse_core_info()
    nw = info.num_cores * info.num_subcores
    assert B % nw == 0 and D == _L
    b_per_w = B // nw
    assert b_per_w % _G == 0
    n_groups = b_per_w // _G
    mesh = plsc.VectorSubcoreMesh(core_axis_name="c", subcore_axis_name="s")

    @functools.partial(
        pl.kernel,
        mesh=mesh,
        out_type=jax.ShapeDtypeStruct((D, B), jnp.float32),
        scratch_types=[
            pltpu.VMEM((b_per_w,), jnp.int32),
            pltpu.VMEM((_G, D, _W), jnp.float32),
            pltpu.VMEM((D, b_per_w), jnp.float32),
            pltpu.SemaphoreType.DMA,
        ],
        compiler_params=pltpu.CompilerParams(
            use_tc_tiling_on_sc=False, needs_layout_passes=False
        ),
    )
    def gather_kernel(tableT_hbm, idx_hbm, outT_hbm, idx_v, ring_v, rows_v, sem):
        wid = lax.axis_index("s") * info.num_cores + lax.axis_index("c")
        base = wid * b_per_w
        pltpu.sync_copy(idx_hbm.at[wid], idx_v)
        lane = lax.iota(jnp.int32, _L)

        def body(g, carry):
            xs = [idx_v[pl.ds(g * _G + t * _L, _L)] for t in range(_G // _L)]
            rs, copies = [], []
            for s in range(_G):
                t, l = divmod(s, _L)
                r = jnp.sum(jnp.where(lane == l, xs[t], 0))
                rs.append(r)
                r0 = pl.multiple_of(lax.bitwise_and(r, -_W), _W)
                copies.append(
                    pltpu.async_copy(
                        tableT_hbm.at[:, pl.ds(r0, _W)],
                        ring_v.at[s],
                        sem,
                    )
                )
            for c in copies:
                c.wait()
            for s in range(_G):
                j = lax.bitwise_and(rs[s], _W - 1)
                vals = plsc.load_gather(
                    ring_v,
                    [
                        jnp.full((_L,), s, jnp.int32),
                        lane,
                        jnp.full((_L,), j, jnp.int32),
                    ],
                )
                plsc.store_scatter(
                    rows_v,
                    [lane, jnp.full((_L,), g * _G + s, jnp.int32)],
                    vals,
                )
            return carry

        lax.fori_loop(0, n_groups, body, 0)
        pltpu.sync_copy(rows_v, outT_hbm.at[:, pl.ds(base, b_per_w)])

    return gather_kernel, nw, b_per_w


def kernel(x, table):
    B = x.shape[0]
    V, D = table.shape
    gather, nw, b_per_w = _make_gather(V, D, B)
    idx = jnp.asarray(x, jnp.int32).reshape(nw, b_per_w)
    return gather(table.T, idx).T


# speedup vs baseline: 2.7943x; 2.7943x over previous
import functools

import jax
import jax.numpy as jnp
from jax import lax
from jax.experimental import pallas as pl
from jax.experimental.pallas import tpu as pltpu
from jax.experimental.pallas import tpu_sc as plsc

_CHUNK = 128


@functools.lru_cache(maxsize=None)
def _make_gather(V, D, B):
    info = plsc.get_sparse_core_info()
    nw = info.num_cores * info.num_subcores
    assert B % nw == 0
    b_per_w = B // nw
    assert b_per_w % _CHUNK == 0
    n_chunks = b_per_w // _CHUNK
    mesh = plsc.VectorSubcoreMesh(core_axis_name="c", subcore_axis_name="s")

    @functools.partial(
        pl.kernel,
        mesh=mesh,
        out_type=jax.ShapeDtypeStruct((B, D), jnp.float32),
        scratch_types=[
            pltpu.VMEM((n_chunks, _CHUNK), jnp.int32),
            pltpu.VMEM((b_per_w, D), jnp.float32),
            pltpu.SemaphoreType.DMA,
        ],
        compiler_params=pltpu.CompilerParams(use_tc_tiling_on_sc=False),
    )
    def gather_kernel(table_hbm, idx_hbm, out_hbm, idx_v, rows_v, sem):
        wid = lax.axis_index("s") * info.num_cores + lax.axis_index("c")
        pltpu.sync_copy(idx_hbm.at[wid], idx_v)
        copies = [
            pltpu.async_copy(
                table_hbm.at[idx_v.at[j]],
                rows_v.at[pl.ds(j * _CHUNK, _CHUNK)],
                sem,
            )
            for j in range(n_chunks)
        ]
        for c in copies:
            c.wait()
        pltpu.sync_copy(rows_v, out_hbm.at[pl.ds(wid * b_per_w, b_per_w)])

    return gather_kernel, nw, n_chunks


def kernel(x, table):
    B = x.shape[0]
    V, D = table.shape
    gather, nw, n_chunks = _make_gather(V, D, B)
    idx = jnp.asarray(x, jnp.int32).reshape(nw, n_chunks, _CHUNK)
    return gather(table, idx)


# speedup vs baseline: 15.3269x; 5.4851x over previous
import functools

import jax
import jax.numpy as jnp
from jax import lax
from jax.experimental import pallas as pl
from jax.experimental.pallas import tpu as pltpu
from jax.experimental.pallas import tpu_sc as plsc

_L = 16
_W = 128
_G = 32


@functools.lru_cache(maxsize=None)
def _make_gather(V, D, B):
    info = plsc.get_sparse_core_info()
    nw = info.num_cores * info.num_subcores
    assert B % nw == 0 and D == _L
    b_per_w = B // nw
    assert b_per_w % _G == 0
    n_groups = b_per_w // _G
    mesh = plsc.VectorSubcoreMesh(core_axis_name="c", subcore_axis_name="s")

    @functools.partial(
        pl.kernel,
        mesh=mesh,
        out_type=jax.ShapeDtypeStruct((D, B), jnp.float32),
        scratch_types=[
            pltpu.VMEM((b_per_w,), jnp.int32),
            pltpu.VMEM((_G, D, _W), jnp.float32),
            pltpu.VMEM((D, b_per_w), jnp.float32),
            pltpu.SemaphoreType.DMA,
        ],
        compiler_params=pltpu.CompilerParams(needs_layout_passes=False),
    )
    def gather_kernel(tableT_hbm, idx_hbm, outT_hbm, idx_v, ring_v, rows_v, sem):
        wid = lax.axis_index("s") * info.num_cores + lax.axis_index("c")
        base = wid * b_per_w
        pltpu.sync_copy(idx_hbm.at[wid], idx_v)
        lane = lax.iota(jnp.int32, _L)

        def body(g, carry):
            xs = [idx_v[pl.ds(g * _G + t * _L, _L)] for t in range(_G // _L)]
            rs, copies = [], []
            for s in range(_G):
                t, l = divmod(s, _L)
                r = jnp.sum(jnp.where(lane == l, xs[t], 0))
                rs.append(r)
                r0 = pl.multiple_of(lax.bitwise_and(r, -_W), _W)
                copies.append(
                    pltpu.async_copy(
                        tableT_hbm.at[:, pl.ds(r0, _W)],
                        ring_v.at[s],
                        sem,
                    )
                )
            for c in copies:
                c.wait()
            for s in range(_G):
                j = lax.bitwise_and(rs[s], _W - 1)
                vals = plsc.load_gather(
                    ring_v,
                    [
                        jnp.full((_L,), s, jnp.int32),
                        lane,
                        jnp.full((_L,), j, jnp.int32),
                    ],
                )
                plsc.store_scatter(
                    rows_v,
                    [lane, jnp.full((_L,), g * _G + s, jnp.int32)],
                    vals,
                )
            return carry

        lax.fori_loop(0, n_groups, body, 0)
        pltpu.sync_copy(rows_v, outT_hbm.at[:, pl.ds(base, b_per_w)])

    return gather_kernel, nw, b_per_w


def kernel(x, table):
    B = x.shape[0]
    V, D = table.shape
    gather, nw, b_per_w = _make_gather(V, D, B)
    idx = jnp.asarray(x, jnp.int32).reshape(nw, b_per_w)
    return gather(table.T, idx).T
